# baseline (speedup 1.0000x reference)
import jax
import jax.numpy as jnp
from jax import lax
from jax.experimental import pallas as pl
from jax.experimental.pallas import tpu as pltpu
from jax.experimental.pallas import tpu_sc as plsc

N_NODES = 10000
N_EDGES = 320000
D = 128

NC = 2
NS = 16
NW = NC * NS

CHUNK = 128
CHUNKS_PER_TILE = 80
E_PAD = NW * CHUNKS_PER_TILE * CHUNK
N_CHUNKS = E_PAD // CHUNK
ACC_ROWS = N_NODES + 8

ROWS_PER_SUB = 624
TAIL_ROWS = N_NODES - ROWS_PER_SUB * NS
TAIL_BASE = ROWS_PER_SUB * NS


def _mm_body(f_ref, wt_ref, n_ref, o_ref):
    o_ref[...] = (
        jnp.dot(f_ref[...], wt_ref[...], preferred_element_type=jnp.float32)
        * n_ref[...]
    )


def _final_body(p_ref, n_ref, o_ref):
    s = p_ref[0] + p_ref[1]
    o_ref[...] = jnp.maximum(s * n_ref[...], 0.0)


def _sc_body(h_hbm, src_hbm, dst_hbm, zero_hbm, out_hbm, idx_s, idx_d, rows, sem, acc):
    cid = lax.axis_index("c")
    sid = lax.axis_index("s")
    wid = sid * NC + cid

    base_rows = sid * ROWS_PER_SUB
    pltpu.sync_copy(
        zero_hbm.at[pl.ds(base_rows, ROWS_PER_SUB)],
        acc.at[pl.ds(base_rows, ROWS_PER_SUB)],
    )

    @pl.when(sid == 0)
    def _():
        pltpu.sync_copy(
            zero_hbm.at[pl.ds(TAIL_BASE, TAIL_ROWS)],
            acc.at[pl.ds(TAIL_BASE, TAIL_ROWS)],
        )

    pltpu.sync_copy(src_hbm.at[pl.ds(wid * CHUNKS_PER_TILE, CHUNKS_PER_TILE)], idx_s)
    pltpu.sync_copy(dst_hbm.at[pl.ds(wid * CHUNKS_PER_TILE, CHUNKS_PER_TILE)], idx_d)

    plsc.subcore_barrier()

    def body(j, carry):
        pltpu.async_copy(h_hbm.at[idx_s.at[j]], rows, sem).wait()
        pltpu.sync_copy(rows, acc.at[idx_d.at[j]], add=True)
        return carry

    lax.fori_loop(0, CHUNKS_PER_TILE, body, 0)

    plsc.subcore_barrier()
    pltpu.sync_copy(
        acc.at[pl.ds(base_rows, ROWS_PER_SUB)],
        out_hbm.at[cid, pl.ds(base_rows, ROWS_PER_SUB)],
    )

    @pl.when(sid == 0)
    def _():
        pltpu.sync_copy(
            acc.at[pl.ds(TAIL_BASE, TAIL_ROWS)],
            out_hbm.at[cid, pl.ds(TAIL_BASE, TAIL_ROWS)],
        )


def _make_sc_call():
    mesh = plsc.VectorSubcoreMesh(core_axis_name="c", subcore_axis_name="s")
    return pl.kernel(
        _sc_body,
        out_type=jax.ShapeDtypeStruct((NC, N_NODES, D), jnp.float32),
        mesh=mesh,
        scratch_types=[
            pltpu.VMEM((CHUNKS_PER_TILE, CHUNK), jnp.int32),
            pltpu.VMEM((CHUNKS_PER_TILE, CHUNK), jnp.int32),
            pltpu.VMEM((CHUNK, D), jnp.float32),
            pltpu.SemaphoreType.DMA,
            pltpu.VMEM_SHARED((ACC_ROWS, D), jnp.float32),
        ],
    )


@jax.jit
def kernel(feature, edge_index, norm, W):
    R = 1000

    h = pl.pallas_call(
        _mm_body,
        grid=(N_NODES // R,),
        in_specs=[
            pl.BlockSpec((R, D), lambda i: (i, 0)),
            pl.BlockSpec((D, D), lambda i: (0, 0)),
            pl.BlockSpec((R, 1), lambda i: (i, 0)),
        ],
        out_specs=pl.BlockSpec((R, D), lambda i: (i, 0)),
        out_shape=jax.ShapeDtypeStruct((N_NODES, D), jnp.float32),
    )(feature, W.T, norm)

    pad = E_PAD - N_EDGES
    src = jnp.concatenate(
        [edge_index[0].astype(jnp.int32), jnp.zeros((pad,), jnp.int32)]
    ).reshape(N_CHUNKS, CHUNK)
    dst = jnp.concatenate(
        [edge_index[1].astype(jnp.int32), jnp.full((pad,), N_NODES, jnp.int32)]
    ).reshape(N_CHUNKS, CHUNK)
    zeros = jnp.zeros((N_NODES, D), jnp.float32)

    partials = _make_sc_call()(h, src, dst, zeros)

    out = pl.pallas_call(
        _final_body,
        grid=(N_NODES // R,),
        in_specs=[
            pl.BlockSpec((NC, R, D), lambda i: (0, i, 0)),
            pl.BlockSpec((R, 1), lambda i: (i, 0)),
        ],
        out_specs=pl.BlockSpec((R, D), lambda i: (i, 0)),
        out_shape=jax.ShapeDtypeStruct((N_NODES, D), jnp.float32),
    )(partials, norm)
    return out


# speedup vs baseline: 2.1018x; 2.1018x over previous
import jax
import jax.numpy as jnp
from jax import lax
from jax.experimental import pallas as pl
from jax.experimental.pallas import tpu as pltpu
from jax.experimental.pallas import tpu_sc as plsc

N_NODES = 10000
N_EDGES = 320000
D = 128

NC = 2
NS = 16
NW = NC * NS

CHUNK = 128
CHUNKS_PER_TILE = 80
E_PAD = NW * CHUNKS_PER_TILE * CHUNK
N_CHUNKS = E_PAD // CHUNK
H_ROWS = N_NODES + 8

ROWS_PER_SUB = 624
TAIL_ROWS = N_NODES - ROWS_PER_SUB * NS
TAIL_BASE = ROWS_PER_SUB * NS


def _mm_body(f_ref, wt_ref, n_ref, o_ref):
    o_ref[...] = (
        jnp.dot(f_ref[...], wt_ref[...], preferred_element_type=jnp.float32)
        * n_ref[...]
    )


def _final_body(p_ref, n_ref, o_ref):
    s = p_ref[0] + p_ref[1]
    o_ref[...] = jnp.maximum(s * n_ref[...], 0.0)


def _sc_body(h_hbm, src_hbm, dst_hbm, zero_hbm, out_hbm, idx_s, idx_d, rows, sem, acc):
    cid = lax.axis_index("c")
    sid = lax.axis_index("s")
    wid = sid * NC + cid

    base_rows = sid * ROWS_PER_SUB
    pltpu.sync_copy(
        zero_hbm.at[pl.ds(base_rows, ROWS_PER_SUB)],
        acc.at[pl.ds(base_rows, ROWS_PER_SUB)],
    )

    @pl.when(sid == 0)
    def _():
        pltpu.sync_copy(
            zero_hbm.at[pl.ds(TAIL_BASE, TAIL_ROWS)],
            acc.at[pl.ds(TAIL_BASE, TAIL_ROWS)],
        )

    pltpu.sync_copy(src_hbm.at[pl.ds(wid * CHUNKS_PER_TILE, CHUNKS_PER_TILE)], idx_s)
    pltpu.sync_copy(dst_hbm.at[pl.ds(wid * CHUNKS_PER_TILE, CHUNKS_PER_TILE)], idx_d)

    plsc.subcore_barrier()

    def body(j, carry):
        pltpu.async_copy(h_hbm.at[idx_s.at[j]], rows, sem).wait()
        pltpu.sync_copy(rows, acc.at[idx_d.at[j]], add=True)
        return carry

    lax.fori_loop(0, CHUNKS_PER_TILE, body, 0)

    plsc.subcore_barrier()
    pltpu.sync_copy(
        acc.at[pl.ds(base_rows, ROWS_PER_SUB)],
        out_hbm.at[cid, pl.ds(base_rows, ROWS_PER_SUB)],
    )

    @pl.when(sid == 0)
    def _():
        pltpu.sync_copy(
            acc.at[pl.ds(TAIL_BASE, TAIL_ROWS)],
            out_hbm.at[cid, pl.ds(TAIL_BASE, TAIL_ROWS)],
        )


def _make_sc_call():
    mesh = plsc.VectorSubcoreMesh(core_axis_name="c", subcore_axis_name="s")
    return pl.kernel(
        _sc_body,
        out_type=jax.ShapeDtypeStruct((NC, N_NODES, D), jnp.float32),
        mesh=mesh,
        scratch_types=[
            pltpu.VMEM((CHUNKS_PER_TILE, CHUNK), jnp.int32),
            pltpu.VMEM((CHUNKS_PER_TILE, CHUNK), jnp.int32),
            pltpu.VMEM((CHUNK, D), jnp.float32),
            pltpu.SemaphoreType.DMA,
            pltpu.VMEM_SHARED((N_NODES, D), jnp.float32),
        ],
    )


@jax.jit
def kernel(feature, edge_index, norm, W):
    R = 1000

    h = pl.pallas_call(
        _mm_body,
        grid=(N_NODES // R,),
        in_specs=[
            pl.BlockSpec((R, D), lambda i: (i, 0)),
            pl.BlockSpec((D, D), lambda i: (0, 0)),
            pl.BlockSpec((R, 1), lambda i: (i, 0)),
        ],
        out_specs=pl.BlockSpec((R, D), lambda i: (i, 0)),
        out_shape=jax.ShapeDtypeStruct((N_NODES, D), jnp.float32),
    )(feature, W.T, norm)

    pad = E_PAD - N_EDGES
    h = jnp.concatenate([h, jnp.zeros((H_ROWS - N_NODES, D), jnp.float32)])
    pad_ids = jnp.arange(pad, dtype=jnp.int32)
    src = jnp.concatenate(
        [edge_index[0].astype(jnp.int32), N_NODES + pad_ids % 8]
    ).reshape(N_CHUNKS, CHUNK)
    dst = jnp.concatenate(
        [edge_index[1].astype(jnp.int32), pad_ids % N_NODES]
    ).reshape(N_CHUNKS, CHUNK)
    zeros = jnp.zeros((N_NODES, D), jnp.float32)

    partials = _make_sc_call()(h, src, dst, zeros)

    out = pl.pallas_call(
        _final_body,
        grid=(N_NODES // R,),
        in_specs=[
            pl.BlockSpec((NC, R, D), lambda i: (0, i, 0)),
            pl.BlockSpec((R, 1), lambda i: (i, 0)),
        ],
        out_specs=pl.BlockSpec((R, D), lambda i: (i, 0)),
        out_shape=jax.ShapeDtypeStruct((N_NODES, D), jnp.float32),
    )(partials, norm)
    return out


# speedup vs baseline: 2.8283x; 1.3457x over previous
import jax
import jax.numpy as jnp
from jax import lax
from jax.experimental import pallas as pl
from jax.experimental.pallas import tpu as pltpu
from jax.experimental.pallas import tpu_sc as plsc

N_NODES = 10000
N_EDGES = 320000
D = 128

NC = 2
NS = 16
NW = NC * NS

CHUNK = 128
CHUNKS_PER_TILE = 80
PASS = CHUNKS_PER_TILE // 2
E_PAD = NW * CHUNKS_PER_TILE * CHUNK
N_CHUNKS = E_PAD // CHUNK
H_ROWS = N_NODES + 8

ROWS_PER_SUB = 624
TAIL_ROWS = N_NODES - ROWS_PER_SUB * NS
TAIL_BASE = ROWS_PER_SUB * NS


def _mm_body(f_ref, wt_ref, n_ref, o_ref):
    o_ref[...] = (
        jnp.dot(f_ref[...], wt_ref[...], preferred_element_type=jnp.float32)
        * n_ref[...]
    )


def _final_body(p_ref, n_ref, o_ref):
    s = p_ref[0] + p_ref[1]
    o_ref[...] = jnp.maximum(s * n_ref[...], 0.0)


def _sc_body(
    h_hbm, src_hbm, dst_hbm, zero_hbm, out_hbm,
    idx_s, idx_d, rows0, rows1, sem0, sem1, acc,
):
    rows = (rows0, rows1)
    sem = (sem0, sem1)
    cid = lax.axis_index("c")
    sid = lax.axis_index("s")
    wid = sid * NC + cid

    base_rows = sid * ROWS_PER_SUB
    pltpu.sync_copy(
        zero_hbm.at[pl.ds(base_rows, ROWS_PER_SUB)],
        acc.at[pl.ds(base_rows, ROWS_PER_SUB)],
    )

    @pl.when(sid == 0)
    def _():
        pltpu.sync_copy(
            zero_hbm.at[pl.ds(TAIL_BASE, TAIL_ROWS)],
            acc.at[pl.ds(TAIL_BASE, TAIL_ROWS)],
        )

    plsc.subcore_barrier()

    for p in (0, 1):
        pass_base = wid * CHUNKS_PER_TILE + p * PASS
        pltpu.sync_copy(src_hbm.at[pl.ds(pass_base, PASS)], idx_s)
        pltpu.sync_copy(dst_hbm.at[pl.ds(pass_base, PASS)], idx_d)

        pltpu.async_copy(h_hbm.at[idx_s.at[0]], rows[0], sem[0])

        def body(i, carry):
            for b in (0, 1):
                j = 2 * i + b
                nb = 1 - b

                @pl.when(j + 1 < PASS)
                def _():
                    pltpu.async_copy(h_hbm.at[idx_s.at[j + 1]], rows[nb], sem[nb])

                pltpu.make_async_copy(
                    h_hbm.at[pl.ds(0, CHUNK)], rows[b], sem[b]
                ).wait()
                pltpu.sync_copy(rows[b], acc.at[idx_d.at[j]], add=True)
            return carry

        lax.fori_loop(0, PASS // 2, body, 0)

    plsc.subcore_barrier()
    pltpu.sync_copy(
        acc.at[pl.ds(base_rows, ROWS_PER_SUB)],
        out_hbm.at[cid, pl.ds(base_rows, ROWS_PER_SUB)],
    )

    @pl.when(sid == 0)
    def _():
        pltpu.sync_copy(
            acc.at[pl.ds(TAIL_BASE, TAIL_ROWS)],
            out_hbm.at[cid, pl.ds(TAIL_BASE, TAIL_ROWS)],
        )


def _make_sc_call():
    mesh = plsc.VectorSubcoreMesh(core_axis_name="c", subcore_axis_name="s")
    return pl.kernel(
        _sc_body,
        out_type=jax.ShapeDtypeStruct((NC, N_NODES, D), jnp.float32),
        mesh=mesh,
        scratch_types=[
            pltpu.VMEM((PASS, CHUNK), jnp.int32),
            pltpu.VMEM((PASS, CHUNK), jnp.int32),
            pltpu.VMEM((CHUNK, D), jnp.float32),
            pltpu.VMEM((CHUNK, D), jnp.float32),
            pltpu.SemaphoreType.DMA,
            pltpu.SemaphoreType.DMA,
            pltpu.VMEM_SHARED((N_NODES, D), jnp.float32),
        ],
    )


@jax.jit
def kernel(feature, edge_index, norm, W):
    R = 1000

    h = pl.pallas_call(
        _mm_body,
        grid=(N_NODES // R,),
        in_specs=[
            pl.BlockSpec((R, D), lambda i: (i, 0)),
            pl.BlockSpec((D, D), lambda i: (0, 0)),
            pl.BlockSpec((R, 1), lambda i: (i, 0)),
        ],
        out_specs=pl.BlockSpec((R, D), lambda i: (i, 0)),
        out_shape=jax.ShapeDtypeStruct((N_NODES, D), jnp.float32),
    )(feature, W.T, norm)

    pad = E_PAD - N_EDGES
    h = jnp.concatenate([h, jnp.zeros((H_ROWS - N_NODES, D), jnp.float32)])
    pad_ids = jnp.arange(pad, dtype=jnp.int32)
    src = jnp.concatenate(
        [edge_index[0].astype(jnp.int32), N_NODES + pad_ids % 8]
    ).reshape(N_CHUNKS, CHUNK)
    dst = jnp.concatenate(
        [edge_index[1].astype(jnp.int32), pad_ids % N_NODES]
    ).reshape(N_CHUNKS, CHUNK)
    zeros = jnp.zeros((N_NODES, D), jnp.float32)

    partials = _make_sc_call()(h, src, dst, zeros)

    out = pl.pallas_call(
        _final_body,
        grid=(N_NODES // R,),
        in_specs=[
            pl.BlockSpec((NC, R, D), lambda i: (0, i, 0)),
            pl.BlockSpec((R, 1), lambda i: (i, 0)),
        ],
        out_specs=pl.BlockSpec((R, D), lambda i: (i, 0)),
        out_shape=jax.ShapeDtypeStruct((N_NODES, D), jnp.float32),
    )(partials, norm)
    return out
